# baseline (speedup 1.0000x reference)
import functools

import jax
import jax.numpy as jnp
from jax import lax
from jax.experimental import pallas as pl
from jax.experimental.pallas import tpu as pltpu
from jax.experimental.pallas import tpu_sc as plsc

_NBUF = 2
_STRIPES = 4


def _sc_gather_coarse(table2, idx):
    B = idx.shape[0]
    info = plsc.get_sparse_core_info()
    nw = info.num_cores * info.num_subcores
    b_per_w = B // nw
    ngroups = b_per_w // 16
    mesh = plsc.VectorSubcoreMesh(core_axis_name="c", subcore_axis_name="s")

    @functools.partial(
        pl.kernel,
        mesh=mesh,
        out_type=jax.ShapeDtypeStruct((B, 128), jnp.float32),
        scratch_types=[
            pltpu.VMEM((b_per_w,), jnp.int32),
            pltpu.VMEM((b_per_w,), jnp.int32),
            pltpu.VMEM((b_per_w, 128), jnp.float32),
            pltpu.SemaphoreType.DMA,
        ],
    )
    def gather_kernel(table_hbm, idx_hbm, out_hbm, idx_v, coarse_v, rows_v,
                      sem):
        wid = lax.axis_index("s") * info.num_cores + lax.axis_index("c")
        base = wid * b_per_w
        pltpu.sync_copy(idx_hbm.at[pl.ds(base, b_per_w)], idx_v)
        for g in range(ngroups):
            v = idx_v[pl.ds(g * 16, 16)]
            coarse_v[pl.ds(g * 16, 16)] = lax.shift_right_logical(v, 3)
        pltpu.async_copy(table_hbm.at[coarse_v], rows_v, sem).wait()
        pltpu.sync_copy(rows_v, out_hbm.at[pl.ds(base, b_per_w)])

    return gather_kernel(table2, idx)


def _select_body(coarse_ref, fine_ref, emb_ref):
    fine = fine_ref[...]
    acc = jnp.zeros(emb_ref.shape, jnp.float32)
    for g in range(8):
        acc = acc + jnp.where(fine == g, coarse_ref[:, g * 16:(g + 1) * 16],
                              0.0)
    emb_ref[...] = acc.astype(jnp.bfloat16)


def _tc_select(coarse, fine):
    B = coarse.shape[0]
    return pl.pallas_call(
        _select_body,
        out_shape=jax.ShapeDtypeStruct((B, 16), jnp.bfloat16),
    )(coarse, fine)


def _make_project_body(B, V, D, b_tile, n_steps):
    rows = b_tile // _STRIPES

    def body(emb_hbm, wt_hbm, b_hbm, out_ref, obuf, emb_v, wt_v, b_v, sems,
             in_sem):
        i = pl.program_id(0)
        slot = lax.rem(i, _NBUF)

        @pl.when(i == 0)
        def _load_inputs():
            pltpu.make_async_copy(emb_hbm, emb_v, in_sem).start()
            pltpu.make_async_copy(emb_hbm, emb_v, in_sem).wait()
            pltpu.make_async_copy(wt_hbm, wt_v, in_sem).start()
            pltpu.make_async_copy(wt_hbm, wt_v, in_sem).wait()
            pltpu.make_async_copy(b_hbm, b_v, in_sem).start()
            pltpu.make_async_copy(b_hbm, b_v, in_sem).wait()

        def stripe_copy(s):
            return pltpu.make_async_copy(
                obuf.at[slot, pl.ds(s * rows, rows), :],
                out_ref.at[pl.ds(i * b_tile + s * rows, rows), :],
                sems.at[slot, s],
            )

        @pl.when(i >= _NBUF)
        def _drain_slot():
            for s in range(_STRIPES):
                stripe_copy(s).wait()

        obuf[slot] = (
            lax.dot_general(
                emb_v[pl.ds(i * b_tile, b_tile), :],
                wt_v[...],
                dimension_numbers=(((1,), (0,)), ((), ())),
                preferred_element_type=jnp.float32,
            )
            + b_v[...]
        )

        for s in range(_STRIPES):
            stripe_copy(s).start()

        @pl.when(i == n_steps - 1)
        def _drain_all():
            for k in range(_NBUF):
                for s in range(_STRIPES):
                    pltpu.make_async_copy(
                        obuf.at[k, pl.ds(s * rows, rows), :],
                        out_ref.at[pl.ds(s * rows, rows), :],
                        sems.at[k, s],
                    ).wait()

    return body


def _tc_project(emb, w_t, b_out, b_tile=64):
    B = emb.shape[0]
    D, V = w_t.shape
    n_steps = B // b_tile
    b2 = b_out.reshape(1, V)
    return pl.pallas_call(
        _make_project_body(B, V, D, b_tile, n_steps),
        grid=(n_steps,),
        in_specs=[
            pl.BlockSpec(memory_space=pl.ANY),
            pl.BlockSpec(memory_space=pl.ANY),
            pl.BlockSpec(memory_space=pl.ANY),
        ],
        out_specs=pl.BlockSpec(memory_space=pl.ANY),
        out_shape=jax.ShapeDtypeStruct((B, V), jnp.float32),
        scratch_shapes=[
            pltpu.VMEM((_NBUF, b_tile, V), jnp.float32),
            pltpu.VMEM((B, D), jnp.bfloat16),
            pltpu.VMEM((D, V), jnp.bfloat16),
            pltpu.VMEM((1, V), jnp.float32),
            pltpu.SemaphoreType.DMA((_NBUF, _STRIPES)),
            pltpu.SemaphoreType.DMA,
        ],
    )(emb, w_t, b2)


def _split_store_body(out_ref, obuf, sems):
    B, V, b_tile = 1024, 100000, 64
    n_steps = B // b_tile
    rows = b_tile // _STRIPES
    aligned = (V // 128) * 128
    tail = V - aligned
    i = pl.program_id(0)
    slot = lax.rem(i, _NBUF)

    def stripe_copies(slot_, row0, s):
        src = obuf.at[slot_, pl.ds(s * rows, rows)]
        dst = out_ref.at[pl.ds(row0 + s * rows, rows)]
        return (
            pltpu.make_async_copy(src.at[:, pl.ds(0, aligned)],
                                  dst.at[:, pl.ds(0, aligned)],
                                  sems.at[slot_, s, 0]),
            pltpu.make_async_copy(src.at[:, pl.ds(aligned, tail)],
                                  dst.at[:, pl.ds(aligned, tail)],
                                  sems.at[slot_, s, 1]),
        )

    @pl.when(i >= _NBUF)
    def _drain_slot():
        for s in range(_STRIPES):
            for c in stripe_copies(slot, 0, s):
                c.wait()

    obuf[slot] = jnp.full((b_tile, V), 1.5, jnp.float32)

    for s in range(_STRIPES):
        for c in stripe_copies(slot, i * b_tile, s):
            c.start()

    @pl.when(i == n_steps - 1)
    def _drain_all():
        for k in range(_NBUF):
            for s in range(_STRIPES):
                for c in stripe_copies(k, 0, s):
                    c.wait()


def kernel(curr, embed_table, W_out, b_out):
    B, V, b_tile = 1024, 100000, 64
    return pl.pallas_call(
        _split_store_body,
        grid=(B // b_tile,),
        out_specs=pl.BlockSpec(memory_space=pl.ANY),
        out_shape=jax.ShapeDtypeStruct((B, V), jnp.float32),
        scratch_shapes=[
            pltpu.VMEM((_NBUF, b_tile, V), jnp.float32),
            pltpu.SemaphoreType.DMA((_NBUF, _STRIPES, 2)),
        ],
    )()


# speedup vs baseline: 2.3820x; 2.3820x over previous
import functools

import jax
import jax.numpy as jnp
from jax import lax
from jax.experimental import pallas as pl
from jax.experimental.pallas import tpu as pltpu
from jax.experimental.pallas import tpu_sc as plsc


def _sc_gather_coarse(table2, idx):
    B = idx.shape[0]
    info = plsc.get_sparse_core_info()
    nw = info.num_cores * info.num_subcores
    b_per_w = B // nw
    ngroups = b_per_w // 16
    mesh = plsc.VectorSubcoreMesh(core_axis_name="c", subcore_axis_name="s")

    @functools.partial(
        pl.kernel,
        mesh=mesh,
        out_type=jax.ShapeDtypeStruct((B, 128), jnp.float32),
        scratch_types=[
            pltpu.VMEM((b_per_w,), jnp.int32),
            pltpu.VMEM((b_per_w,), jnp.int32),
            pltpu.VMEM((b_per_w, 128), jnp.float32),
            pltpu.SemaphoreType.DMA,
        ],
    )
    def gather_kernel(table_hbm, idx_hbm, out_hbm, idx_v, coarse_v, rows_v,
                      sem):
        wid = lax.axis_index("s") * info.num_cores + lax.axis_index("c")
        base = wid * b_per_w
        pltpu.sync_copy(idx_hbm.at[pl.ds(base, b_per_w)], idx_v)
        for g in range(ngroups):
            v = idx_v[pl.ds(g * 16, 16)]
            coarse_v[pl.ds(g * 16, 16)] = lax.shift_right_logical(v, 3)
        pltpu.async_copy(table_hbm.at[coarse_v], rows_v, sem).wait()
        pltpu.sync_copy(rows_v, out_hbm.at[pl.ds(base, b_per_w)])

    return gather_kernel(table2, idx)


def _select_body(coarse_ref, fine_ref, emb_ref):
    fine = fine_ref[...]
    B = fine.shape[0]
    acc = jnp.zeros((B, 16), jnp.float32)
    for g in range(8):
        acc = acc + jnp.where(fine == g, coarse_ref[:, g * 16:(g + 1) * 16],
                              0.0)
    emb_ref[0:16, :] = acc.T.astype(jnp.bfloat16)
    emb_ref[16:17, :] = jnp.ones((1, B), jnp.bfloat16)


def _tc_select(coarse, fine):
    B = coarse.shape[0]
    return pl.pallas_call(
        _select_body,
        out_shape=jax.ShapeDtypeStruct((17, B), jnp.bfloat16),
    )(coarse, fine)


def _project_body(w_ref, emb_ref, out_ref):
    out_ref[...] = lax.dot_general(
        w_ref[...],
        emb_ref[...],
        dimension_numbers=(((0,), (0,)), ((), ())),
        preferred_element_type=jnp.float32,
    )


def _tc_project_t(emb_aug, w_aug, V, n_tile=2048):
    K, B = emb_aug.shape
    Vp = w_aug.shape[1]
    grid = (Vp // n_tile,)
    return pl.pallas_call(
        _project_body,
        grid=grid,
        in_specs=[
            pl.BlockSpec((K, n_tile), lambda i: (0, i)),
            pl.BlockSpec((K, B), lambda i: (0, 0)),
        ],
        out_specs=pl.BlockSpec((n_tile, B), lambda i: (i, 0)),
        out_shape=jax.ShapeDtypeStruct((V, B), jnp.float32),
    )(w_aug, emb_aug)


def kernel(curr, embed_table, W_out, b_out):
    curr = curr.astype(jnp.int32)
    V, D = embed_table.shape
    table2 = embed_table.reshape(V // 8, 128)
    coarse_rows = _sc_gather_coarse(table2, curr)
    fine = (curr & 7).reshape(-1, 1)
    emb_aug = _tc_select(coarse_rows, fine)
    w_aug = jnp.concatenate(
        [W_out.T.astype(jnp.bfloat16), b_out[None, :].astype(jnp.bfloat16)], 0
    )
    Vp = ((V + 2047) // 2048) * 2048
    w_aug = jnp.pad(w_aug, ((0, 0), (0, Vp - V)))
    out_t = _tc_project_t(emb_aug, w_aug, V)
    return out_t.T


# speedup vs baseline: 2.5302x; 1.0622x over previous
import functools

import jax
import jax.numpy as jnp
from jax import lax
from jax.experimental import pallas as pl
from jax.experimental.pallas import tpu as pltpu
from jax.experimental.pallas import tpu_sc as plsc


def _sc_gather_coarse(table2, idx):
    B = idx.shape[0]
    info = plsc.get_sparse_core_info()
    nw = info.num_cores * info.num_subcores
    b_per_w = B // nw
    ngroups = b_per_w // 16
    mesh = plsc.VectorSubcoreMesh(core_axis_name="c", subcore_axis_name="s")

    @functools.partial(
        pl.kernel,
        mesh=mesh,
        out_type=jax.ShapeDtypeStruct((B, 128), jnp.float32),
        scratch_types=[
            pltpu.VMEM((b_per_w,), jnp.int32),
            pltpu.VMEM((b_per_w,), jnp.int32),
            pltpu.VMEM((b_per_w, 128), jnp.float32),
            pltpu.SemaphoreType.DMA,
        ],
    )
    def gather_kernel(table_hbm, idx_hbm, out_hbm, idx_v, coarse_v, rows_v,
                      sem):
        wid = lax.axis_index("s") * info.num_cores + lax.axis_index("c")
        base = wid * b_per_w
        pltpu.sync_copy(idx_hbm.at[pl.ds(base, b_per_w)], idx_v)
        for g in range(ngroups):
            v = idx_v[pl.ds(g * 16, 16)]
            coarse_v[pl.ds(g * 16, 16)] = lax.shift_right_logical(v, 3)
        pltpu.async_copy(table_hbm.at[coarse_v], rows_v, sem).wait()
        pltpu.sync_copy(rows_v, out_hbm.at[pl.ds(base, b_per_w)])

    return gather_kernel(table2, idx)


def _select_body(coarse_ref, fine_ref, emb_ref):
    fine = fine_ref[...]
    B = fine.shape[0]
    acc = jnp.zeros((B, 16), jnp.float32)
    for g in range(8):
        acc = acc + jnp.where(fine == g, coarse_ref[:, g * 16:(g + 1) * 16],
                              0.0)
    emb_ref[0:16, :] = acc.T.astype(jnp.bfloat16)
    emb_ref[16:17, :] = jnp.ones((1, B), jnp.bfloat16)


def _tc_select(coarse, fine):
    B = coarse.shape[0]
    return pl.pallas_call(
        _select_body,
        out_shape=jax.ShapeDtypeStruct((17, B), jnp.bfloat16),
    )(coarse, fine)


def _project_body(w_ref, emb_ref, out_ref):
    out_ref[...] = lax.dot_general(
        w_ref[...],
        emb_ref[...],
        dimension_numbers=(((0,), (0,)), ((), ())),
        preferred_element_type=jnp.float32,
    )


def _tc_project_t(emb_aug, w_aug, V, n_tile=4096):
    K, B = emb_aug.shape
    Vp = w_aug.shape[1]
    grid = (Vp // n_tile,)
    return pl.pallas_call(
        _project_body,
        grid=grid,
        in_specs=[
            pl.BlockSpec((K, n_tile), lambda i: (0, i)),
            pl.BlockSpec((K, B), lambda i: (0, 0)),
        ],
        out_specs=pl.BlockSpec((n_tile, B), lambda i: (i, 0)),
        out_shape=jax.ShapeDtypeStruct((V, B), jnp.float32),
    )(w_aug, emb_aug)


def kernel(curr, embed_table, W_out, b_out):
    curr = curr.astype(jnp.int32)
    V, D = embed_table.shape
    table2 = (
        embed_table.T.reshape(D, V // 8, 8)
        .transpose(1, 2, 0)
        .reshape(V // 8, 8 * D)
    )
    coarse_rows = _sc_gather_coarse(table2, curr)
    fine = (curr & 7).reshape(-1, 1)
    emb_aug = _tc_select(coarse_rows, fine)
    w_aug = jnp.concatenate(
        [W_out.T.astype(jnp.bfloat16), b_out[None, :].astype(jnp.bfloat16)], 0
    )
    Vp = ((V + 4095) // 4096) * 4096
    w_aug = jnp.pad(w_aug, ((0, 0), (0, Vp - V)))
    out_t = _tc_project_t(emb_aug, w_aug, V)
    return out_t.T


# speedup vs baseline: 2.9835x; 1.1791x over previous
import functools

import jax
import jax.numpy as jnp
from jax import lax
from jax.experimental import pallas as pl
from jax.experimental.pallas import tpu as pltpu
from jax.experimental.pallas import tpu_sc as plsc


def _sc_gather_t(table_flat, idx, V, D):
    B = idx.shape[0]
    info = plsc.get_sparse_core_info()
    nw = info.num_cores * info.num_subcores
    b_per_w = B // nw
    ngroups = b_per_w // 16
    mesh = plsc.VectorSubcoreMesh(core_axis_name="c", subcore_axis_name="s")

    @functools.partial(
        pl.kernel,
        mesh=mesh,
        out_type=jax.ShapeDtypeStruct((nw, D * b_per_w), jnp.float32),
        scratch_types=[
            pltpu.VMEM((b_per_w,), jnp.int32),
            pltpu.VMEM((D * b_per_w,), jnp.int32),
            pltpu.VMEM((D * b_per_w,), jnp.float32),
            pltpu.SemaphoreType.DMA,
        ],
    )
    def gather_kernel(table_hbm, idx_hbm, out_hbm, idx_v, fidx_v, vals_v, sem):
        wid = lax.axis_index("s") * info.num_cores + lax.axis_index("c")
        base = wid * b_per_w
        pltpu.sync_copy(idx_hbm.at[pl.ds(base, b_per_w)], idx_v)
        for k in range(D):
            for g in range(ngroups):
                v = idx_v[pl.ds(g * 16, 16)]
                fidx_v[pl.ds(k * b_per_w + g * 16, 16)] = v + k * V
        pltpu.async_copy(table_hbm.at[fidx_v], vals_v, sem).wait()
        pltpu.sync_copy(vals_v, out_hbm.at[wid])

    out3 = gather_kernel(table_flat, idx)
    return (
        out3.reshape(nw, D, b_per_w).transpose(1, 0, 2).reshape(D, B)
    )


def _project_body(w_ref, emb_ref, out_ref):
    B = emb_ref.shape[1]
    emb_aug = jnp.concatenate(
        [emb_ref[...].astype(jnp.bfloat16), jnp.ones((1, B), jnp.bfloat16)], 0
    )
    out_ref[...] = lax.dot_general(
        w_ref[...],
        emb_aug,
        dimension_numbers=(((0,), (0,)), ((), ())),
        preferred_element_type=jnp.float32,
    )


def _tc_project_t(emb_t, w_aug, V, n_tile=4096):
    D, B = emb_t.shape
    K, Vp = w_aug.shape
    grid = (Vp // n_tile,)
    return pl.pallas_call(
        _project_body,
        grid=grid,
        in_specs=[
            pl.BlockSpec((K, n_tile), lambda i: (0, i)),
            pl.BlockSpec((D, B), lambda i: (0, 0)),
        ],
        out_specs=pl.BlockSpec((n_tile, B), lambda i: (i, 0)),
        out_shape=jax.ShapeDtypeStruct((V, B), jnp.float32),
    )(w_aug, emb_t)


def kernel(curr, embed_table, W_out, b_out):
    curr = curr.astype(jnp.int32)
    V, D = embed_table.shape
    table_flat = embed_table.T.reshape(V * D)
    emb_t = _sc_gather_t(table_flat, curr, V, D)
    w_aug = jnp.concatenate(
        [W_out.T.astype(jnp.bfloat16), b_out[None, :].astype(jnp.bfloat16)], 0
    )
    Vp = ((V + 4095) // 4096) * 4096
    w_aug = jnp.pad(w_aug, ((0, 0), (0, Vp - V)))
    out_t = _tc_project_t(emb_t, w_aug, V)
    return out_t.T
